# baseline (speedup 1.0000x reference)
import functools

import jax
import jax.numpy as jnp
import numpy as np
from jax import lax
from jax.experimental import pallas as pl
from jax.experimental.pallas import tpu as pltpu
from jax.experimental.pallas import tpu_sc as plsc

_N = 16384
_D = 128
_K = 64

_IDX = np.array(
    [16183, 8472, 4286, 739, 9083, 15353, 9849, 12308, 13717, 1495, 10730,
     10881, 683, 7946, 10144, 2116, 12896, 9193, 2401, 13873, 16161, 14668,
     7696, 9805, 14673, 9586, 5488, 5278, 9423, 14991, 118, 12454, 5346,
     10704, 6339, 8211, 1867, 3984, 2082, 4575, 15817, 15266, 14173, 5664,
     5852, 11042, 11497, 6940, 207, 2756, 14070, 7812, 8376, 1814, 4486,
     4559, 12120, 14755, 2691, 12986, 6945, 11910, 1512, 7341],
    dtype=np.int32,
)

_mesh = plsc.ScalarSubcoreMesh(axis_name="c", num_cores=2)


@functools.partial(
    pl.kernel,
    mesh=_mesh,
    out_type=jax.ShapeDtypeStruct((_K, _D), jnp.float32),
    scratch_types=[
        pltpu.SemaphoreType.DMA,
    ],
)
def _gather_rows(x_hbm, out_hbm, sem):
    cid = lax.axis_index("c")
    half = _K // 2
    for c in range(2):

        @pl.when(cid == c)
        def _(c=c):
            copies = [
                pltpu.async_copy(
                    x_hbm.at[pl.ds(int(_IDX[i]), 1)],
                    out_hbm.at[pl.ds(i, 1)],
                    sem,
                )
                for i in range(c * half, (c + 1) * half)
            ]
            for cp in copies:
                cp.wait()


def kernel(x):
    return _gather_rows(x)
